# Initial kernel scaffold; baseline (speedup 1.0000x reference)
import functools

import jax
import jax.numpy as jnp
from jax import lax
from jax.experimental import pallas as pl
from jax.experimental.pallas import tpu as pltpu
from jax.experimental.pallas import tpu_sc as plsc

NC = 2
NS = 16
LANES = 16
CHUNK = 128


def _cdiv(a, b):
    return (a + b - 1) // b


@functools.lru_cache(maxsize=None)
def _make_spmm(n, d, n_chunks):
    assert n % NS == 0, "node count must split evenly across tiles"
    rows_per_tile = n // NS
    full = rows_per_tile // CHUNK
    rem = rows_per_tile - full * CHUNK
    mesh = plsc.VectorSubcoreMesh(core_axis_name="c", subcore_axis_name="s",
                                  num_cores=NC, num_subcores=NS)

    def body(src_hbm, dst_hbm, w_hbm, h_hbm, out_hbm,
             srcb, dstb, wb, rows, acc, lsem, gsem):
        c = lax.axis_index("c")
        s = lax.axis_index("s")

        d1 = pltpu.async_copy(src_hbm.at[c, s], srcb, lsem)
        d2 = pltpu.async_copy(dst_hbm.at[c, s], dstb, lsem)
        d3 = pltpu.async_copy(w_hbm.at[c, s], wb, lsem)

        zv = jnp.zeros((LANES,), jnp.float32)

        def zrow(j, carry):
            for k in range(d // LANES):
                rows[j, pl.ds(k * LANES, LANES)] = zv
            return carry

        lax.fori_loop(0, CHUNK, zrow, 0)
        base = s * rows_per_tile
        for i in range(full):
            pltpu.sync_copy(rows, acc.at[pl.ds(base + i * CHUNK, CHUNK)])
        if rem:
            pltpu.sync_copy(rows.at[pl.ds(0, rem)],
                            acc.at[pl.ds(base + full * CHUNK, rem)])

        d1.wait()
        d2.wait()
        d3.wait()
        plsc.subcore_barrier()

        def chunk_body(b, carry):
            pltpu.async_copy(h_hbm.at[srcb.at[b]], rows, gsem).wait()

            def edge(j, carry2):
                wj = plsc.load_gather(
                    wb, [jnp.full((LANES,), b, jnp.int32),
                         jnp.full((LANES,), j, jnp.int32)])
                for k in range(d // LANES):
                    sl = pl.ds(k * LANES, LANES)
                    rows[j, sl] = rows[j, sl] * wj
                return carry2

            lax.fori_loop(0, CHUNK, edge, 0)
            pltpu.sync_copy(rows, acc.at[dstb.at[b]], add=True)
            return carry

        lax.fori_loop(0, n_chunks, chunk_body, 0)
        plsc.subcore_barrier()

        for i in range(full + (1 if rem else 0)):
            sz = CHUNK if i < full else rem
            off = base + i * CHUNK
            pltpu.sync_copy(acc.at[pl.ds(off, sz)], rows.at[pl.ds(0, sz)])
            pltpu.sync_copy(rows.at[pl.ds(0, sz)], out_hbm.at[c, pl.ds(off, sz)])

    return pl.kernel(
        body,
        out_type=jax.ShapeDtypeStruct((NC, n, d), jnp.float32),
        mesh=mesh,
        scratch_types=[
            pltpu.VMEM((n_chunks, CHUNK), jnp.int32),
            pltpu.VMEM((n_chunks, CHUNK), jnp.int32),
            pltpu.VMEM((n_chunks, CHUNK), jnp.float32),
            pltpu.VMEM((CHUNK, d), jnp.float32),
            pltpu.VMEM_SHARED((n, d), jnp.float32),
            pltpu.SemaphoreType.DMA,
            pltpu.SemaphoreType.DMA,
        ],
    )


def _pad_edges(src, dst, w, n):
    e = src.shape[0]
    per = NC * NS * CHUNK
    n_chunks = _cdiv(e, per)
    e_pad = n_chunks * per
    pad = e_pad - e
    if pad:
        fill = jnp.arange(pad, dtype=jnp.int32) % n
        src = jnp.concatenate([src, fill])
        dst = jnp.concatenate([dst, fill])
        w = jnp.concatenate([w, jnp.zeros((pad,), w.dtype)])
    shape = (NC, NS, n_chunks, CHUNK)
    return src.reshape(shape), dst.reshape(shape), w.reshape(shape), n_chunks


def _mm1(x, w1):
    n = x.shape[0]
    dh = w1.shape[1]

    def body(x_ref, w_ref, o_ref):
        o_ref[...] = jnp.dot(x_ref[...], w_ref[...],
                             preferred_element_type=jnp.float32)

    return pl.pallas_call(
        body, out_shape=jax.ShapeDtypeStruct((n, dh), jnp.float32))(x, w1)


def _fuse2(p, w2):
    n = p.shape[1]
    dout = w2.shape[1]

    def body(p_ref, w_ref, o_ref):
        h = jnp.maximum(p_ref[0] + p_ref[1], 0.0)
        o_ref[...] = jnp.dot(h, w_ref[...], preferred_element_type=jnp.float32)

    return pl.pallas_call(
        body, out_shape=jax.ShapeDtypeStruct((n, dout), jnp.float32))(p, w2)


def _final(p):
    n, dout = p.shape[1], p.shape[2]

    def body(p_ref, o_ref):
        o_ref[...] = jnp.maximum(p_ref[0] + p_ref[1], 0.0)

    return pl.pallas_call(
        body, out_shape=jax.ShapeDtypeStruct((n, dout), jnp.float32))(p)


def kernel(x, edge_index, edge_weight, W1, W2):
    n = x.shape[0]
    srcp, dstp, wp, n_chunks = _pad_edges(
        edge_index[0], edge_index[1], edge_weight, n)
    spmm_h = _make_spmm(n, W1.shape[1], n_chunks)
    spmm_o = _make_spmm(n, W2.shape[1], n_chunks)

    h = _mm1(x, W1)
    p1 = spmm_h(srcp, dstp, wp, h)
    h2 = _fuse2(p1, W2)
    p2 = spmm_o(srcp, dstp, wp, h2)
    return _final(p2)


# speedup vs baseline: 6.3671x; 6.3671x over previous
import functools

import jax
import jax.numpy as jnp
from jax import lax
from jax.experimental import pallas as pl
from jax.experimental.pallas import tpu as pltpu
from jax.experimental.pallas import tpu_sc as plsc

NC = 2
NS = 16
LANES = 16
CHUNK = 128


def _cdiv(a, b):
    return (a + b - 1) // b


def _rows_per_tile(n):
    return _cdiv(_cdiv(n, NS), 8) * 8


@functools.lru_cache(maxsize=None)
def _make_spmm(n_pad, d, n_chunks):
    rows_per_tile = n_pad // NS
    full = rows_per_tile // CHUNK
    rem = rows_per_tile - full * CHUNK
    mesh = plsc.VectorSubcoreMesh(core_axis_name="c", subcore_axis_name="s",
                                  num_cores=NC, num_subcores=NS)

    def body(src_hbm, dst_hbm, w_hbm, h_hbm, out_hbm,
             srcb, dstb, wb, rows, acc, lsem, gsem):
        c = lax.axis_index("c")
        s = lax.axis_index("s")

        d1 = pltpu.async_copy(src_hbm.at[c, s], srcb, lsem)
        d2 = pltpu.async_copy(dst_hbm.at[c, s], dstb, lsem)
        d3 = pltpu.async_copy(w_hbm.at[c, s], wb, lsem)

        zv = jnp.zeros((LANES,), jnp.float32)

        def zrow(j, carry):
            for k in range(d // LANES):
                rows[j, pl.ds(k * LANES, LANES)] = zv
            return carry

        lax.fori_loop(0, CHUNK, zrow, 0)
        base = s * rows_per_tile
        for i in range(full):
            pltpu.sync_copy(rows, acc.at[pl.ds(base + i * CHUNK, CHUNK)])
        if rem:
            pltpu.sync_copy(rows.at[pl.ds(0, rem)],
                            acc.at[pl.ds(base + full * CHUNK, rem)])

        d1.wait()
        d2.wait()
        d3.wait()
        plsc.subcore_barrier()

        def chunk_body(b, carry):
            pltpu.async_copy(h_hbm.at[srcb.at[b]], rows, gsem).wait()

            def wgroup(g, carry2):
                wv = wb[b, pl.ds(g * LANES, LANES)]
                for jj in range(LANES):
                    j = g * LANES + jj
                    wj = wv[jj]
                    for k in range(d // LANES):
                        sl = pl.ds(k * LANES, LANES)
                        rows[j, sl] = rows[j, sl] * wj
                return carry2

            lax.fori_loop(0, CHUNK // LANES, wgroup, 0)
            pltpu.sync_copy(rows, acc.at[dstb.at[b]], add=True)
            return carry

        lax.fori_loop(0, n_chunks, chunk_body, 0)
        plsc.subcore_barrier()

        for i in range(full + (1 if rem else 0)):
            sz = CHUNK if i < full else rem
            off = base + i * CHUNK
            pltpu.sync_copy(acc.at[pl.ds(off, sz)], rows.at[pl.ds(0, sz)])
            pltpu.sync_copy(rows.at[pl.ds(0, sz)], out_hbm.at[c, pl.ds(off, sz)])

    return pl.kernel(
        body,
        out_type=jax.ShapeDtypeStruct((NC, n_pad, d), jnp.float32),
        mesh=mesh,
        compiler_params=pltpu.CompilerParams(use_tc_tiling_on_sc=False),
        scratch_types=[
            pltpu.VMEM((n_chunks, CHUNK), jnp.int32),
            pltpu.VMEM((n_chunks, CHUNK), jnp.int32),
            pltpu.VMEM((n_chunks, CHUNK), jnp.float32),
            pltpu.VMEM((CHUNK, d), jnp.float32),
            pltpu.VMEM_SHARED((n_pad, d), jnp.float32),
            pltpu.SemaphoreType.DMA,
            pltpu.SemaphoreType.DMA,
        ],
    )


def _pad_edges(src, dst, w, n):
    e = src.shape[0]
    per = NC * NS * CHUNK
    n_chunks = _cdiv(e, per)
    e_pad = n_chunks * per
    pad = e_pad - e
    if pad:
        fill = jnp.arange(pad, dtype=jnp.int32) % n
        src = jnp.concatenate([src, fill])
        dst = jnp.concatenate([dst, fill])
        w = jnp.concatenate([w, jnp.zeros((pad,), w.dtype)])
    shape = (NC, NS, n_chunks, CHUNK)
    return src.reshape(shape), dst.reshape(shape), w.reshape(shape), n_chunks


def _mm1(x, w1):
    n = x.shape[0]
    dh = w1.shape[1]

    def body(x_ref, w_ref, o_ref):
        o_ref[...] = jnp.dot(x_ref[...], w_ref[...],
                             preferred_element_type=jnp.float32)

    return pl.pallas_call(
        body, out_shape=jax.ShapeDtypeStruct((n, dh), jnp.float32))(x, w1)


def _fuse2(p, w2):
    n = p.shape[1]
    dout = w2.shape[1]

    def body(p_ref, w_ref, o_ref):
        h = jnp.maximum(p_ref[0] + p_ref[1], 0.0)
        o_ref[...] = jnp.dot(h, w_ref[...], preferred_element_type=jnp.float32)

    return pl.pallas_call(
        body, out_shape=jax.ShapeDtypeStruct((n, dout), jnp.float32))(p, w2)


def _final(p, n):
    dout = p.shape[2]

    def body(p_ref, o_ref):
        o_ref[...] = jnp.maximum(p_ref[0, :n] + p_ref[1, :n], 0.0)

    return pl.pallas_call(
        body, out_shape=jax.ShapeDtypeStruct((n, dout), jnp.float32))(p)


def kernel(x, edge_index, edge_weight, W1, W2):
    n = x.shape[0]
    n_pad = _rows_per_tile(n) * NS
    srcp, dstp, wp, n_chunks = _pad_edges(
        edge_index[0], edge_index[1], edge_weight, n)
    spmm_h = _make_spmm(n_pad, W1.shape[1], n_chunks)
    spmm_o = _make_spmm(n_pad, W2.shape[1], n_chunks)

    h = _mm1(x, W1)
    p1 = spmm_h(srcp, dstp, wp, h)
    h2 = _fuse2(p1, W2)
    p2 = spmm_o(srcp, dstp, wp, h2)
    return _final(p2, n)
